# Initial kernel scaffold; baseline (speedup 1.0000x reference)
import jax
import jax.numpy as jnp
from jax.experimental import pallas as pl


def kernel(points, params):
    raise NotImplementedError("write your pallas kernel here")


# speedup vs baseline: 1.0878x; 1.0878x over previous
import functools

import jax
import jax.numpy as jnp
from jax.experimental import pallas as pl


def _linear_body(x_ref, w_ref, o_ref):
    o_ref[...] = jnp.dot(x_ref[...], w_ref[...],
                         preferred_element_type=jnp.float32)


def _pallas_linear(x2d, W):
    R, Cin = x2d.shape
    Cout = W.shape[1]
    TR = 1024
    while R % TR != 0:
        TR //= 2
    return pl.pallas_call(
        _linear_body,
        grid=(R // TR,),
        in_specs=[
            pl.BlockSpec((TR, Cin), lambda i: (i, 0)),
            pl.BlockSpec((Cin, Cout), lambda i: (0, 0)),
        ],
        out_specs=pl.BlockSpec((TR, Cout), lambda i: (i, 0)),
        out_shape=jax.ShapeDtypeStruct((R, Cout), jnp.float32),
    )(x2d, W)


def _bn(x, gamma, beta):
    axes = tuple(range(x.ndim - 1))
    mu = jnp.mean(x, axis=axes, keepdims=True)
    var = jnp.var(x, axis=axes, keepdims=True)
    return gamma * (x - mu) / jnp.sqrt(var + 1e-5) + beta


def _mlp(x, layers):
    shp = x.shape
    x2 = x.reshape(-1, shp[-1])
    for l in layers:
        y = _pallas_linear(x2, l["W"])
        y = _bn(y, l["gamma"], l["beta"])
        x2 = jnp.maximum(y, 0.0)
    return x2.reshape(shp[:-1] + (x2.shape[-1],))


def _index_points(p, idx):
    return jax.vmap(lambda pb, ib: pb[ib])(p, idx)


def _sqdist(a, b):
    return (jnp.sum(a ** 2, -1)[:, :, None] + jnp.sum(b ** 2, -1)[:, None, :]
            - 2.0 * (a @ b.transpose(0, 2, 1)))


def _fps(xyz, M):
    xyz = jax.lax.stop_gradient(xyz)
    Bb, Nn, _ = xyz.shape

    def body(i, st):
        idx, dist, far = st
        idx = idx.at[:, i].set(far)
        cent = jnp.take_along_axis(xyz, far[:, None, None].astype(jnp.int32), axis=1)
        d = jnp.sum((xyz - cent) ** 2, axis=-1)
        dist = jnp.minimum(dist, d)
        far = jnp.argmax(dist, axis=-1).astype(jnp.int32)
        return idx, dist, far

    st = (jnp.zeros((Bb, M), jnp.int32),
          jnp.full((Bb, Nn), 1e10, jnp.float32),
          jnp.zeros((Bb,), jnp.int32))
    idx, _, _ = jax.lax.fori_loop(0, M, body, st)
    return idx


def _ball_query(radius, K, xyz, new_xyz):
    sqd = _sqdist(jax.lax.stop_gradient(new_xyz), jax.lax.stop_gradient(xyz))
    Nn = xyz.shape[1]
    gidx = jnp.broadcast_to(jnp.arange(Nn), sqd.shape)
    gidx = jnp.where(sqd > radius * radius, Nn, gidx)
    vals, _ = jax.lax.top_k(-gidx.astype(jnp.float32), K)
    gidx = (-vals).astype(jnp.int32)
    first = gidx[:, :, :1]
    return jnp.where(gidx == Nn, first, gidx)


NUM_CENTROIDS_ = (5120, 1024, 256)
RADIUS_ = (0.02, 0.08, 0.2)
NUM_NEIGHBOURS_ = (32, 32, 32)
NUM_FP_NEIGHBOURS_ = (3, 3, 3)


def _sa(xyz, feat, layers, M, radius, K):
    fidx = _fps(xyz, M)
    new_xyz = _index_points(xyz, fidx)
    gidx = _ball_query(radius, K, xyz, new_xyz)
    gxyz = _index_points(xyz, gidx) - new_xyz[:, :, None, :]
    gfeat = jnp.concatenate([gxyz, _index_points(feat, gidx)], axis=-1)
    out = _mlp(gfeat, layers)
    return new_xyz, jnp.max(out, axis=2)


def _fp(dxyz, sxyz, dfeat, sfeat, layers, k):
    sqd = _sqdist(dxyz, sxyz)
    negd, idx = jax.lax.top_k(-sqd, k)
    d = jnp.maximum(-negd, 0.0)
    w = 1.0 / (d + 1e-8)
    w = w / jnp.sum(w, axis=-1, keepdims=True)
    interp = jnp.sum(_index_points(sfeat, idx) * w[..., None], axis=2)
    x = jnp.concatenate([dfeat, interp], axis=-1)
    return _mlp(x, layers)


def kernel(points, params):
    xyz = points[:, :3, :].transpose(0, 2, 1)
    feat = points[:, 3:, :].transpose(0, 2, 1)
    ixyz = [xyz]
    ifeat = [feat]
    for i in range(3):
        xyz, feat = _sa(xyz, feat, params["sa"][i], NUM_CENTROIDS_[i],
                        RADIUS_[i], NUM_NEIGHBOURS_[i])
        ixyz.append(xyz)
        ifeat.append(feat)
    sxyz, sfeat = xyz, feat
    for i in range(3):
        dxyz = ixyz[-2 - i]
        dfeat = ifeat[-2 - i]
        sfeat = _fp(dxyz, sxyz, dfeat, sfeat, params["fp"][i], NUM_FP_NEIGHBOURS_[i])
        sxyz = dxyz
    x = _mlp(sfeat, params["seg"])
    xs = _pallas_linear(x.reshape(-1, x.shape[-1]), params["score_W"])
    xs = xs.reshape(x.shape[:-1] + (1,)) + params["score_b"]
    xs = _bn(xs, params["score_gamma"], params["score_beta"])
    xs = jax.nn.sigmoid(xs)
    return sfeat.transpose(0, 2, 1), xs.reshape(points.shape[0], points.shape[2], -1)


# speedup vs baseline: 1.6914x; 1.5549x over previous
import functools

import jax
import jax.numpy as jnp
from jax.experimental import pallas as pl
from jax.experimental.pallas import tpu as pltpu


NUM_CENTROIDS_ = (5120, 1024, 256)
RADIUS_ = (0.02, 0.08, 0.2)
NUM_NEIGHBOURS_ = (32, 32, 32)
NUM_FP_NEIGHBOURS_ = (3, 3, 3)


def _pick_tr(R):
    TR = 1024
    while R % TR != 0:
        TR //= 2
    return TR


def _bnrelu_body(y_ref, mu_ref, den_ref, g_ref, b_ref, o_ref):
    xn = g_ref[...] * (y_ref[...] - mu_ref[...]) / den_ref[...] + b_ref[...]
    o_ref[...] = jnp.maximum(xn, 0.0)


def _bnrelu_max_body(K, y_ref, mu_ref, den_ref, g_ref, b_ref, o_ref):
    xn = g_ref[...] * (y_ref[...] - mu_ref[...]) / den_ref[...] + b_ref[...]
    x = jnp.maximum(xn, 0.0)
    TR, C = x.shape
    o_ref[...] = jnp.max(x.reshape(TR // K, K, C), axis=1)


def _bnrelu(y_nd, g, b, maxpool_k=None):
    C = y_nd.shape[-1]
    axes = tuple(range(y_nd.ndim - 1))
    mu = jnp.mean(y_nd, axis=axes).reshape(1, C)
    den = jnp.sqrt(jnp.var(y_nd, axis=axes) + 1e-5).reshape(1, C)
    y = y_nd.reshape(-1, C)
    R = y.shape[0]
    TR = _pick_tr(R)
    specs = [
        pl.BlockSpec((TR, C), lambda i: (i, 0)),
        pl.BlockSpec((1, C), lambda i: (0, 0)),
        pl.BlockSpec((1, C), lambda i: (0, 0)),
        pl.BlockSpec((1, C), lambda i: (0, 0)),
        pl.BlockSpec((1, C), lambda i: (0, 0)),
    ]
    if maxpool_k is None:
        out = pl.pallas_call(
            _bnrelu_body,
            grid=(R // TR,),
            in_specs=specs,
            out_specs=pl.BlockSpec((TR, C), lambda i: (i, 0)),
            out_shape=jax.ShapeDtypeStruct((R, C), jnp.float32),
        )(y, mu, den, g.reshape(1, -1), b.reshape(1, -1))
        return out.reshape(y_nd.shape)
    K = maxpool_k
    out = pl.pallas_call(
        functools.partial(_bnrelu_max_body, K),
        grid=(R // TR,),
        in_specs=specs,
        out_specs=pl.BlockSpec((TR // K, C), lambda i: (i, 0)),
        out_shape=jax.ShapeDtypeStruct((R // K, C), jnp.float32),
    )(y, mu, den, g.reshape(1, -1), b.reshape(1, -1))
    return out.reshape(y_nd.shape[:-2] + (C,))


def _mlp(x, layers, maxpool_k=None):
    n = len(layers)
    for j, l in enumerate(layers):
        y = x @ l["W"]
        k = maxpool_k if j == n - 1 else None
        x = _bnrelu(y, l["gamma"], l["beta"], maxpool_k=k)
    return x


def _fps_body(M, N, xyz_ref, idx_ref, dist_ref):
    NR = N // 128
    dist_ref[...] = jnp.full((NR, 128), 1e10, jnp.float32)
    lin = (jax.lax.broadcasted_iota(jnp.int32, (NR, 128), 0) * 128
           + jax.lax.broadcasted_iota(jnp.int32, (NR, 128), 1))
    lane = jax.lax.broadcasted_iota(jnp.int32, (1, 128), 1)

    def body(i, st):
        far, buf = st
        pos = jnp.remainder(i, 128)
        buf = jnp.where(lane == pos, far, buf)

        @pl.when(pos == 127)
        def _():
            start = pl.multiple_of((i // 128) * 128, 128)
            idx_ref[0, :, pl.ds(start, 128)] = buf

        r = far // 128
        c = far - r * 128
        row = xyz_ref[0, :, pl.ds(r, 1), :]
        sel = lane[0] == c
        cx = jnp.sum(jnp.where(sel, row[0, 0], 0.0))
        cy = jnp.sum(jnp.where(sel, row[1, 0], 0.0))
        cz = jnp.sum(jnp.where(sel, row[2, 0], 0.0))
        dx = xyz_ref[0, 0] - cx
        dy = xyz_ref[0, 1] - cy
        dz = xyz_ref[0, 2] - cz
        d = (dx * dx + dy * dy) + dz * dz
        dist = jnp.minimum(dist_ref[...], d)
        dist_ref[...] = dist
        m = jnp.max(dist)
        return jnp.min(jnp.where(dist == m, lin, N)), buf

    jax.lax.fori_loop(0, M, body,
                      (jnp.int32(0), jnp.zeros((1, 128), jnp.int32)))


def _fps_pallas(xyz, M):
    B, N, _ = xyz.shape
    NR = N // 128
    x4 = xyz.transpose(0, 2, 1).reshape(B, 3, NR, 128)
    out = pl.pallas_call(
        functools.partial(_fps_body, M, N),
        grid=(B,),
        in_specs=[pl.BlockSpec((1, 3, NR, 128), lambda b: (b, 0, 0, 0))],
        out_specs=pl.BlockSpec((1, 1, M), lambda b: (b, 0, 0)),
        out_shape=jax.ShapeDtypeStruct((B, 1, M), jnp.int32),
        scratch_shapes=[pltpu.VMEM((NR, 128), jnp.float32)],
    )(x4)
    return out[:, 0, :]


def _fps(xyz, M):
    return _fps_pallas(xyz, M)


def _index_points(p, idx):
    return jax.vmap(lambda pb, ib: pb[ib])(p, idx)


def _sqdist(a, b):
    return (jnp.sum(a ** 2, -1)[:, :, None] + jnp.sum(b ** 2, -1)[:, None, :]
            - 2.0 * (a @ b.transpose(0, 2, 1)))


def _ball_query(radius, K, xyz, new_xyz):
    sqd = _sqdist(jax.lax.stop_gradient(new_xyz), jax.lax.stop_gradient(xyz))
    Nn = xyz.shape[1]
    gidx = jnp.broadcast_to(jnp.arange(Nn), sqd.shape)
    gidx = jnp.where(sqd > radius * radius, Nn, gidx)
    vals, _ = jax.lax.top_k(-gidx.astype(jnp.float32), K)
    gidx = (-vals).astype(jnp.int32)
    first = gidx[:, :, :1]
    return jnp.where(gidx == Nn, first, gidx)


def _sa(xyz, feat, layers, M, radius, K):
    fidx = _fps(xyz, M)
    new_xyz = _index_points(xyz, fidx)
    gidx = _ball_query(radius, K, xyz, new_xyz)
    gxyz = _index_points(xyz, gidx) - new_xyz[:, :, None, :]
    gfeat = jnp.concatenate([gxyz, _index_points(feat, gidx)], axis=-1)
    return new_xyz, _mlp(gfeat, layers, maxpool_k=K)


def _fp(dxyz, sxyz, dfeat, sfeat, layers, k):
    sqd = _sqdist(dxyz, sxyz)
    negd, idx = jax.lax.top_k(-sqd, k)
    d = jnp.maximum(-negd, 0.0)
    w = 1.0 / (d + 1e-8)
    w = w / jnp.sum(w, axis=-1, keepdims=True)
    interp = jnp.sum(_index_points(sfeat, idx) * w[..., None], axis=2)
    x = jnp.concatenate([dfeat, interp], axis=-1)
    return _mlp(x, layers)


def kernel(points, params):
    B, _, N = points.shape
    xyz = points[:, :3, :].transpose(0, 2, 1)
    feat = points[:, 3:, :].transpose(0, 2, 1)
    ixyz = [xyz]
    ifeat = [feat]
    for i in range(3):
        xyz, feat = _sa(xyz, feat, params["sa"][i], NUM_CENTROIDS_[i],
                        RADIUS_[i], NUM_NEIGHBOURS_[i])
        ixyz.append(xyz)
        ifeat.append(feat)
    sxyz, sfeat = xyz, feat
    for i in range(3):
        dxyz = ixyz[-2 - i]
        dfeat = ifeat[-2 - i]
        sfeat = _fp(dxyz, sxyz, dfeat, sfeat, params["fp"][i],
                    NUM_FP_NEIGHBOURS_[i])
        sxyz = dxyz
    x = _mlp(sfeat, params["seg"])
    xs = x @ params["score_W"] + params["score_b"]
    mu = jnp.mean(xs)
    var = jnp.var(xs)
    xs = (params["score_gamma"] * (xs - mu) / jnp.sqrt(var + 1e-5)
          + params["score_beta"])
    xs = jax.nn.sigmoid(xs)
    return (sfeat.transpose(0, 2, 1), xs.reshape(B, N, 1))
